# Initial kernel scaffold; baseline (speedup 1.0000x reference)
import jax
import jax.numpy as jnp
from jax.experimental import pallas as pl


def kernel(x, edge_index, W1, b1, W2, b2, W3, b3, Wp, bp):
    raise NotImplementedError("write your pallas kernel here")


# speedup vs baseline: 14.5272x; 14.5272x over previous
import functools

import jax
import jax.numpy as jnp
from jax import lax
from jax.experimental import pallas as pl
from jax.experimental.pallas import tpu as pltpu
from jax.experimental.pallas import tpu_sc as plsc

B = 128
NTILES = 32
DEGW = 16


def _pad_rows(n):
  per = -(-n // NTILES)
  per = -(-per // 8) * 8
  return per * NTILES


def _make_deg_kernel(Np, nchunks):
  mesh = plsc.VectorSubcoreMesh(core_axis_name="c", subcore_axis_name="s")
  per_core = nchunks // 2
  per_tile = per_core // 16
  rows_per_tile = Np // NTILES * 2
  zrep = rows_per_tile // B

  @functools.partial(
      pl.kernel,
      mesh=mesh,
      out_type=jax.ShapeDtypeStruct((2, Np, DEGW), jnp.float32),
      compiler_params=pltpu.CompilerParams(use_tc_tiling_on_sc=False),
      scratch_types=[
          pltpu.VMEM((B,), jnp.int32),
          pltpu.VMEM((B, DEGW), jnp.float32),
          pltpu.VMEM((B, DEGW), jnp.float32),
          pltpu.VMEM_SHARED((Np, DEGW), jnp.float32),
      ],
  )
  def k(dst2d_hbm, out_hbm, dst_v, ones_v, stage_v, acc_sh):
    c = lax.axis_index("c")
    s = lax.axis_index("s")

    def fill(i, _):
      ones_v[i, :] = jnp.full((DEGW,), 1.0, jnp.float32)
      stage_v[i, :] = jnp.zeros((DEGW,), jnp.float32)
      return 0

    lax.fori_loop(0, B, fill, 0)

    base_row = s * rows_per_tile
    for r in range(zrep):
      pltpu.sync_copy(stage_v, acc_sh.at[pl.ds(base_row + r * B, B)])
    plsc.subcore_barrier()

    def body(t, _):
      chunk = c * per_core + s * per_tile + t
      pltpu.sync_copy(dst2d_hbm.at[chunk], dst_v)
      pltpu.sync_copy(ones_v, acc_sh.at[dst_v], add=True)
      return 0

    lax.fori_loop(0, per_tile, body, 0)
    plsc.subcore_barrier()

    for r in range(zrep):
      row = base_row + r * B
      pltpu.sync_copy(acc_sh.at[pl.ds(row, B)], stage_v)
      pltpu.sync_copy(stage_v, out_hbm.at[c, pl.ds(row, B)])

  return k


def _make_prop_kernel(Np, nchunks, F):
  mesh = plsc.VectorSubcoreMesh(core_axis_name="c", subcore_axis_name="s")
  per_core = nchunks // 2
  per_tile = per_core // 16
  rows_per_tile = Np // NTILES * 2
  zrep = rows_per_tile // B

  @functools.partial(
      pl.kernel,
      mesh=mesh,
      out_type=jax.ShapeDtypeStruct((2, Np, F), jnp.float32),
      compiler_params=pltpu.CompilerParams(use_tc_tiling_on_sc=False),
      scratch_types=[
          pltpu.VMEM((B,), jnp.int32),
          pltpu.VMEM((B,), jnp.int32),
          pltpu.VMEM((B, F), jnp.float32),
          pltpu.VMEM((B, F), jnp.float32),
          pltpu.VMEM_SHARED((Np, F), jnp.float32),
          pltpu.SemaphoreType.DMA,
      ],
  )
  def k(h_hbm, src2d_hbm, dst2d_hbm, out_hbm, src_v, dst_v, rows_v, stage_v,
        acc_sh, sem):
    c = lax.axis_index("c")
    s = lax.axis_index("s")

    def fill(i, _):
      for j in range(F // 16):
        stage_v[i, pl.ds(j * 16, 16)] = jnp.zeros((16,), jnp.float32)
      return 0

    lax.fori_loop(0, B, fill, 0)

    base_row = s * rows_per_tile
    for r in range(zrep):
      pltpu.sync_copy(stage_v, acc_sh.at[pl.ds(base_row + r * B, B)])
    plsc.subcore_barrier()

    def body(t, _):
      chunk = c * per_core + s * per_tile + t
      pltpu.sync_copy(src2d_hbm.at[chunk], src_v)
      pltpu.sync_copy(dst2d_hbm.at[chunk], dst_v)
      pltpu.async_copy(h_hbm.at[src_v], rows_v, sem).wait()
      pltpu.sync_copy(rows_v, acc_sh.at[dst_v], add=True)
      return 0

    lax.fori_loop(0, per_tile, body, 0)
    plsc.subcore_barrier()

    for r in range(zrep):
      row = base_row + r * B
      pltpu.sync_copy(acc_sh.at[pl.ds(row, B)], stage_v)
      pltpu.sync_copy(stage_v, out_hbm.at[c, pl.ds(row, B)])

  return k


RB = 512


def _tc1_body(x_ref, w_ref, deg_ref, h_ref, dis_ref):
  deg = 1.0 + deg_ref[0, :, 0] + deg_ref[1, :, 0]
  dis = lax.rsqrt(deg)
  h = jnp.dot(x_ref[...], w_ref[...], preferred_element_type=jnp.float32)
  h_ref[...] = h * dis[:, None]
  dis_ref[...] = dis[:, None]


def _tc1(x_pad, W1, degp, Np):
  D = x_pad.shape[1]
  F = W1.shape[1]
  grid = (Np // RB,)
  return pl.pallas_call(
      _tc1_body,
      grid=grid,
      in_specs=[
          pl.BlockSpec((RB, D), lambda i: (i, 0)),
          pl.BlockSpec((D, F), lambda i: (0, 0)),
          pl.BlockSpec((2, RB, DEGW), lambda i: (0, i, 0)),
      ],
      out_specs=[
          pl.BlockSpec((RB, F), lambda i: (i, 0)),
          pl.BlockSpec((RB, 1), lambda i: (i, 0)),
      ],
      out_shape=[
          jax.ShapeDtypeStruct((Np, F), jnp.float32),
          jax.ShapeDtypeStruct((Np, 1), jnp.float32),
      ],
  )(x_pad, W1, degp)


def _tc_mid_body(p_ref, h_ref, dis_ref, b_ref, w_ref, o_ref):
  dis = dis_ref[...]
  z = (p_ref[0] + p_ref[1] + h_ref[...]) * dis + b_ref[...][None, :]
  g = jnp.where(z >= 0, z, 0.01 * z)
  o_ref[...] = jnp.dot(g, w_ref[...],
                       preferred_element_type=jnp.float32) * dis


def _tc_mid(p, h, dis, b, W, Np):
  F = h.shape[1]
  F2 = W.shape[1]
  grid = (Np // RB,)
  return pl.pallas_call(
      _tc_mid_body,
      grid=grid,
      in_specs=[
          pl.BlockSpec((2, RB, F), lambda i: (0, i, 0)),
          pl.BlockSpec((RB, F), lambda i: (i, 0)),
          pl.BlockSpec((RB, 1), lambda i: (i, 0)),
          pl.BlockSpec((F,), lambda i: (0,)),
          pl.BlockSpec((F, F2), lambda i: (0, 0)),
      ],
      out_specs=pl.BlockSpec((RB, F2), lambda i: (i, 0)),
      out_shape=jax.ShapeDtypeStruct((Np, F2), jnp.float32),
  )(p, h, dis, b, W)


def _tc_fin_body(p_ref, h_ref, dis_ref, b_ref, w_ref, bp_ref, o_ref):
  dis = dis_ref[...]
  z = (p_ref[0] + p_ref[1] + h_ref[...]) * dis + b_ref[...][None, :]
  g = jnp.where(z >= 0, z, 0.01 * z)
  o_ref[...] = jnp.dot(g, w_ref[...],
                       preferred_element_type=jnp.float32) + bp_ref[...][None, :]


def _tc_fin(p, h, dis, b, Wp, bp, N):
  F = h.shape[1]
  C = Wp.shape[1]
  rb = 400
  grid = (N // rb,)
  return pl.pallas_call(
      _tc_fin_body,
      grid=grid,
      in_specs=[
          pl.BlockSpec((2, rb, F), lambda i: (0, i, 0)),
          pl.BlockSpec((rb, F), lambda i: (i, 0)),
          pl.BlockSpec((rb, 1), lambda i: (i, 0)),
          pl.BlockSpec((F,), lambda i: (0,)),
          pl.BlockSpec((F, C), lambda i: (0, 0)),
          pl.BlockSpec((C,), lambda i: (0,)),
      ],
      out_specs=pl.BlockSpec((rb, C), lambda i: (i, 0)),
      out_shape=jax.ShapeDtypeStruct((N, C), jnp.float32),
  )(p, h, dis, b, Wp, bp)


@jax.jit
def kernel(x, edge_index, W1, b1, W2, b2, W3, b3, Wp, bp):
  N, D = x.shape
  E = edge_index.shape[1]
  Np = _pad_rows(N)

  epb = NTILES * B
  Ep = -(-E // epb) * epb
  pad_e = Ep - E
  nchunks = Ep // B

  src = jnp.concatenate(
      [edge_index[0], jnp.full((pad_e,), N, jnp.int32)]).reshape(nchunks, B)
  dst = jnp.concatenate(
      [edge_index[1], jnp.full((pad_e,), N, jnp.int32)]).reshape(nchunks, B)

  x_pad = jnp.pad(x, ((0, Np - N), (0, 0)))

  degp = _make_deg_kernel(Np, nchunks)(dst)
  h1, dis = _tc1(x_pad, W1, degp, Np)

  p1 = _make_prop_kernel(Np, nchunks, h1.shape[1])(h1, src, dst)
  h2 = _tc_mid(p1, h1, dis, b1, W2, Np)

  p2 = _make_prop_kernel(Np, nchunks, h2.shape[1])(h2, src, dst)
  h3 = _tc_mid(p2, h2, dis, b2, W3, Np)

  p3 = _make_prop_kernel(Np, nchunks, h3.shape[1])(h3, src, dst)
  out = _tc_fin(p3, h3, dis, b3, Wp, bp, N)
  return out


# speedup vs baseline: 20.1540x; 1.3873x over previous
import functools

import jax
import jax.numpy as jnp
from jax import lax
from jax.experimental import pallas as pl
from jax.experimental.pallas import tpu as pltpu
from jax.experimental.pallas import tpu_sc as plsc

B = 128
NTILES = 32
DEGW = 16


def _pad_rows(n):
  per = -(-n // NTILES)
  per = -(-per // 8) * 8
  return per * NTILES


def _make_deg_kernel(Np, nchunks):
  mesh = plsc.VectorSubcoreMesh(core_axis_name="c", subcore_axis_name="s")
  per_core = nchunks // 2
  per_tile = per_core // 16
  rows_per_tile = Np // NTILES * 2
  zrep = rows_per_tile // B

  @functools.partial(
      pl.kernel,
      mesh=mesh,
      out_type=jax.ShapeDtypeStruct((2, Np, DEGW), jnp.float32),
      compiler_params=pltpu.CompilerParams(use_tc_tiling_on_sc=False),
      scratch_types=[
          pltpu.VMEM((nchunks // NTILES, B), jnp.int32),
          pltpu.VMEM((B, DEGW), jnp.float32),
          pltpu.VMEM((B, DEGW), jnp.float32),
          pltpu.VMEM_SHARED((Np, DEGW), jnp.float32),
          pltpu.SemaphoreType.DMA,
      ],
  )
  def k(dst2d_hbm, out_hbm, dst_all, ones_v, stage_v, acc_sh, sem):
    c = lax.axis_index("c")
    s = lax.axis_index("s")
    base_chunk = c * per_core + s * per_tile
    pltpu.sync_copy(dst2d_hbm.at[pl.ds(base_chunk, per_tile)], dst_all)

    def fill(i, _):
      ones_v[i, :] = jnp.full((DEGW,), 1.0, jnp.float32)
      stage_v[i, :] = jnp.zeros((DEGW,), jnp.float32)
      return 0

    lax.fori_loop(0, B, fill, 0)

    base_row = s * rows_per_tile
    for r in range(zrep):
      pltpu.sync_copy(stage_v, acc_sh.at[pl.ds(base_row + r * B, B)])
    plsc.subcore_barrier()

    def body(t, _):
      pltpu.sync_copy(ones_v, acc_sh.at[dst_all.at[t]], add=True)
      return 0

    lax.fori_loop(0, per_tile, body, 0)
    plsc.subcore_barrier()

    for r in range(zrep):
      row = base_row + r * B
      pltpu.sync_copy(acc_sh.at[pl.ds(row, B)], stage_v)
      pltpu.sync_copy(stage_v, out_hbm.at[c, pl.ds(row, B)])

  return k


def _make_prop_kernel(Np, nchunks, F):
  mesh = plsc.VectorSubcoreMesh(core_axis_name="c", subcore_axis_name="s")
  per_core = nchunks // 2
  per_tile = per_core // 16
  rows_per_tile = Np // NTILES * 2
  zrep = rows_per_tile // B

  @functools.partial(
      pl.kernel,
      mesh=mesh,
      out_type=jax.ShapeDtypeStruct((2, Np, F), jnp.float32),
      compiler_params=pltpu.CompilerParams(use_tc_tiling_on_sc=False),
      scratch_types=[
          pltpu.VMEM((per_tile, B), jnp.int32),
          pltpu.VMEM((per_tile, B), jnp.int32),
          pltpu.VMEM((B, F), jnp.float32),
          pltpu.VMEM((B, F), jnp.float32),
          pltpu.VMEM_SHARED((Np, F), jnp.float32),
          pltpu.SemaphoreType.DMA,
      ],
  )
  def k(h_hbm, src2d_hbm, dst2d_hbm, out_hbm, src_all, dst_all, rows_v,
        stage_v, acc_sh, g0):
    c = lax.axis_index("c")
    s = lax.axis_index("s")
    base_chunk = c * per_core + s * per_tile
    pltpu.sync_copy(src2d_hbm.at[pl.ds(base_chunk, per_tile)], src_all)
    pltpu.sync_copy(dst2d_hbm.at[pl.ds(base_chunk, per_tile)], dst_all)

    def fill(i, _):
      for j in range(F // 16):
        stage_v[i, pl.ds(j * 16, 16)] = jnp.zeros((16,), jnp.float32)
      return 0

    lax.fori_loop(0, B, fill, 0)

    base_row = s * rows_per_tile
    for r in range(zrep):
      pltpu.sync_copy(stage_v, acc_sh.at[pl.ds(base_row + r * B, B)])
    plsc.subcore_barrier()

    def body(t, _):
      pltpu.async_copy(h_hbm.at[src_all.at[t]], rows_v, g0).wait()
      pltpu.async_copy(
          rows_v, acc_sh.at[dst_all.at[t]], g0, add=True).wait()
      return 0

    lax.fori_loop(0, per_tile, body, 0)
    plsc.subcore_barrier()

    for r in range(zrep):
      row = base_row + r * B
      pltpu.sync_copy(acc_sh.at[pl.ds(row, B)], stage_v)
      pltpu.sync_copy(stage_v, out_hbm.at[c, pl.ds(row, B)])

  return k


RB = 512


def _tc1_body(x_ref, w_ref, deg_ref, h_ref, dis_ref):
  deg = 1.0 + deg_ref[0, :, 0] + deg_ref[1, :, 0]
  dis = lax.rsqrt(deg)
  h = jnp.dot(x_ref[...], w_ref[...], preferred_element_type=jnp.float32)
  h_ref[...] = h * dis[:, None]
  dis_ref[...] = dis[:, None]


def _tc1(x_pad, W1, degp, Np):
  D = x_pad.shape[1]
  F = W1.shape[1]
  grid = (Np // RB,)
  return pl.pallas_call(
      _tc1_body,
      grid=grid,
      in_specs=[
          pl.BlockSpec((RB, D), lambda i: (i, 0)),
          pl.BlockSpec((D, F), lambda i: (0, 0)),
          pl.BlockSpec((2, RB, DEGW), lambda i: (0, i, 0)),
      ],
      out_specs=[
          pl.BlockSpec((RB, F), lambda i: (i, 0)),
          pl.BlockSpec((RB, 1), lambda i: (i, 0)),
      ],
      out_shape=[
          jax.ShapeDtypeStruct((Np, F), jnp.float32),
          jax.ShapeDtypeStruct((Np, 1), jnp.float32),
      ],
  )(x_pad, W1, degp)


def _tc_mid_body(p_ref, h_ref, dis_ref, b_ref, w_ref, o_ref):
  dis = dis_ref[...]
  z = (p_ref[0] + p_ref[1] + h_ref[...]) * dis + b_ref[...][None, :]
  g = jnp.where(z >= 0, z, 0.01 * z)
  o_ref[...] = jnp.dot(g, w_ref[...],
                       preferred_element_type=jnp.float32) * dis


def _tc_mid(p, h, dis, b, W, Np):
  F = h.shape[1]
  F2 = W.shape[1]
  grid = (Np // RB,)
  return pl.pallas_call(
      _tc_mid_body,
      grid=grid,
      in_specs=[
          pl.BlockSpec((2, RB, F), lambda i: (0, i, 0)),
          pl.BlockSpec((RB, F), lambda i: (i, 0)),
          pl.BlockSpec((RB, 1), lambda i: (i, 0)),
          pl.BlockSpec((F,), lambda i: (0,)),
          pl.BlockSpec((F, F2), lambda i: (0, 0)),
      ],
      out_specs=pl.BlockSpec((RB, F2), lambda i: (i, 0)),
      out_shape=jax.ShapeDtypeStruct((Np, F2), jnp.float32),
  )(p, h, dis, b, W)


def _tc_fin_body(p_ref, h_ref, dis_ref, b_ref, w_ref, bp_ref, o_ref):
  dis = dis_ref[...]
  z = (p_ref[0] + p_ref[1] + h_ref[...]) * dis + b_ref[...][None, :]
  g = jnp.where(z >= 0, z, 0.01 * z)
  o_ref[...] = jnp.dot(g, w_ref[...],
                       preferred_element_type=jnp.float32) + bp_ref[...][None, :]


def _tc_fin(p, h, dis, b, Wp, bp, N):
  F = h.shape[1]
  C = Wp.shape[1]
  rb = 400
  grid = (N // rb,)
  return pl.pallas_call(
      _tc_fin_body,
      grid=grid,
      in_specs=[
          pl.BlockSpec((2, rb, F), lambda i: (0, i, 0)),
          pl.BlockSpec((rb, F), lambda i: (i, 0)),
          pl.BlockSpec((rb, 1), lambda i: (i, 0)),
          pl.BlockSpec((F,), lambda i: (0,)),
          pl.BlockSpec((F, C), lambda i: (0, 0)),
          pl.BlockSpec((C,), lambda i: (0,)),
      ],
      out_specs=pl.BlockSpec((rb, C), lambda i: (i, 0)),
      out_shape=jax.ShapeDtypeStruct((N, C), jnp.float32),
  )(p, h, dis, b, Wp, bp)


@jax.jit
def kernel(x, edge_index, W1, b1, W2, b2, W3, b3, Wp, bp):
  N, D = x.shape
  E = edge_index.shape[1]
  Np = _pad_rows(N)

  epb = NTILES * B
  Ep = -(-E // epb) * epb
  pad_e = Ep - E
  nchunks = Ep // B

  src = jnp.concatenate(
      [edge_index[0], jnp.full((pad_e,), N, jnp.int32)]).reshape(nchunks, B)
  dst = jnp.concatenate(
      [edge_index[1], jnp.full((pad_e,), N, jnp.int32)]).reshape(nchunks, B)

  x_pad = jnp.pad(x, ((0, Np - N), (0, 0)))

  degp = _make_deg_kernel(Np, nchunks)(dst)
  h1, dis = _tc1(x_pad, W1, degp, Np)

  p1 = _make_prop_kernel(Np, nchunks, h1.shape[1])(h1, src, dst)
  h2 = _tc_mid(p1, h1, dis, b1, W2, Np)

  p2 = _make_prop_kernel(Np, nchunks, h2.shape[1])(h2, src, dst)
  h3 = _tc_mid(p2, h2, dis, b2, W3, Np)

  p3 = _make_prop_kernel(Np, nchunks, h3.shape[1])(h3, src, dst)
  out = _tc_fin(p3, h3, dis, b3, Wp, bp, N)
  return out


# speedup vs baseline: 29.2454x; 1.4511x over previous
import functools

import jax
import jax.numpy as jnp
from jax import lax
from jax.experimental import pallas as pl
from jax.experimental.pallas import tpu as pltpu
from jax.experimental.pallas import tpu_sc as plsc

B = 128
NTILES = 32
DEGW = 16


def _pad_rows(n):
  per = -(-n // NTILES)
  per = -(-per // 8) * 8
  return per * NTILES


def _make_deg_kernel(Np, nchunks):
  mesh = plsc.VectorSubcoreMesh(core_axis_name="c", subcore_axis_name="s")
  per_core = nchunks // 2
  per_tile = per_core // 16
  rows_per_tile = Np // NTILES * 2
  zrep = rows_per_tile // B

  @functools.partial(
      pl.kernel,
      mesh=mesh,
      out_type=jax.ShapeDtypeStruct((2, Np, DEGW), jnp.float32),
      compiler_params=pltpu.CompilerParams(use_tc_tiling_on_sc=False),
      scratch_types=[
          pltpu.VMEM((nchunks // NTILES, B), jnp.int32),
          pltpu.VMEM((B, DEGW), jnp.float32),
          pltpu.VMEM((B, DEGW), jnp.float32),
          pltpu.VMEM_SHARED((Np, DEGW), jnp.float32),
          pltpu.SemaphoreType.DMA,
      ],
  )
  def k(dst2d_hbm, out_hbm, dst_all, ones_v, stage_v, acc_sh, sem):
    c = lax.axis_index("c")
    s = lax.axis_index("s")
    base_chunk = c * per_core + s * per_tile
    pltpu.sync_copy(dst2d_hbm.at[pl.ds(base_chunk, per_tile)], dst_all)

    def fill(i, _):
      ones_v[i, :] = jnp.full((DEGW,), 1.0, jnp.float32)
      stage_v[i, :] = jnp.zeros((DEGW,), jnp.float32)
      return 0

    lax.fori_loop(0, B, fill, 0)

    base_row = s * rows_per_tile
    for r in range(zrep):
      pltpu.sync_copy(stage_v, acc_sh.at[pl.ds(base_row + r * B, B)])
    plsc.subcore_barrier()

    def body(t, _):
      pltpu.sync_copy(ones_v, acc_sh.at[dst_all.at[t]], add=True)
      return 0

    lax.fori_loop(0, per_tile, body, 0)
    plsc.subcore_barrier()

    for r in range(zrep):
      row = base_row + r * B
      pltpu.sync_copy(acc_sh.at[pl.ds(row, B)], stage_v)
      pltpu.sync_copy(stage_v, out_hbm.at[c, pl.ds(row, B)])

  return k


def _make_prop_kernel(Np, nchunks, F):
  mesh = plsc.VectorSubcoreMesh(core_axis_name="c", subcore_axis_name="s")
  per_core = nchunks // 2
  per_tile = per_core // 16
  rows_per_tile = Np // NTILES * 2
  zrep = rows_per_tile // B

  @functools.partial(
      pl.kernel,
      mesh=mesh,
      out_type=jax.ShapeDtypeStruct((2, Np, F), jnp.float32),
      compiler_params=pltpu.CompilerParams(use_tc_tiling_on_sc=False),
      scratch_types=[
          pltpu.VMEM((per_tile, B), jnp.int32),
          pltpu.VMEM((per_tile, B), jnp.int32),
          pltpu.VMEM((B, F), jnp.float32),
          pltpu.VMEM((B, F), jnp.float32),
          pltpu.VMEM_SHARED((Np, F), jnp.float32),
          pltpu.VMEM_SHARED((Np, F), jnp.float32),
          pltpu.SemaphoreType.DMA,
      ],
  )
  def k(h_hbm, src2d_hbm, dst2d_hbm, out_hbm, src_all, dst_all, rows_v,
        stage_v, acc_sh, table_sh, g0):
    c = lax.axis_index("c")
    s = lax.axis_index("s")
    base_chunk = c * per_core + s * per_tile
    pltpu.sync_copy(src2d_hbm.at[pl.ds(base_chunk, per_tile)], src_all)
    pltpu.sync_copy(dst2d_hbm.at[pl.ds(base_chunk, per_tile)], dst_all)
    base_row0 = s * rows_per_tile
    for r in range(zrep):
      row = base_row0 + r * B
      pltpu.sync_copy(h_hbm.at[pl.ds(row, B)], table_sh.at[pl.ds(row, B)])

    def fill(i, _):
      for j in range(F // 16):
        stage_v[i, pl.ds(j * 16, 16)] = jnp.zeros((16,), jnp.float32)
      return 0

    lax.fori_loop(0, B, fill, 0)

    base_row = s * rows_per_tile
    for r in range(zrep):
      pltpu.sync_copy(stage_v, acc_sh.at[pl.ds(base_row + r * B, B)])
    plsc.subcore_barrier()

    def body(t, _):
      pltpu.async_copy(table_sh.at[src_all.at[t]], rows_v, g0).wait()
      pltpu.async_copy(
          rows_v, acc_sh.at[dst_all.at[t]], g0, add=True).wait()
      return 0

    lax.fori_loop(0, per_tile, body, 0)
    plsc.subcore_barrier()

    for r in range(zrep):
      row = base_row + r * B
      pltpu.sync_copy(acc_sh.at[pl.ds(row, B)], stage_v)
      pltpu.sync_copy(stage_v, out_hbm.at[c, pl.ds(row, B)])

  return k


RB = 512


def _tc1_body(x_ref, w_ref, deg_ref, h_ref, dis_ref):
  deg = 1.0 + deg_ref[0, :, 0] + deg_ref[1, :, 0]
  dis = lax.rsqrt(deg)
  h = jnp.dot(x_ref[...], w_ref[...], preferred_element_type=jnp.float32)
  h_ref[...] = h * dis[:, None]
  dis_ref[...] = dis[:, None]


def _tc1(x_pad, W1, degp, Np):
  D = x_pad.shape[1]
  F = W1.shape[1]
  grid = (Np // RB,)
  return pl.pallas_call(
      _tc1_body,
      grid=grid,
      in_specs=[
          pl.BlockSpec((RB, D), lambda i: (i, 0)),
          pl.BlockSpec((D, F), lambda i: (0, 0)),
          pl.BlockSpec((2, RB, DEGW), lambda i: (0, i, 0)),
      ],
      out_specs=[
          pl.BlockSpec((RB, F), lambda i: (i, 0)),
          pl.BlockSpec((RB, 1), lambda i: (i, 0)),
      ],
      out_shape=[
          jax.ShapeDtypeStruct((Np, F), jnp.float32),
          jax.ShapeDtypeStruct((Np, 1), jnp.float32),
      ],
  )(x_pad, W1, degp)


def _tc_mid_body(p_ref, h_ref, dis_ref, b_ref, w_ref, o_ref):
  dis = dis_ref[...]
  z = (p_ref[0] + p_ref[1] + h_ref[...]) * dis + b_ref[...][None, :]
  g = jnp.where(z >= 0, z, 0.01 * z)
  o_ref[...] = jnp.dot(g, w_ref[...],
                       preferred_element_type=jnp.float32) * dis


def _tc_mid(p, h, dis, b, W, Np):
  F = h.shape[1]
  F2 = W.shape[1]
  grid = (Np // RB,)
  return pl.pallas_call(
      _tc_mid_body,
      grid=grid,
      in_specs=[
          pl.BlockSpec((2, RB, F), lambda i: (0, i, 0)),
          pl.BlockSpec((RB, F), lambda i: (i, 0)),
          pl.BlockSpec((RB, 1), lambda i: (i, 0)),
          pl.BlockSpec((F,), lambda i: (0,)),
          pl.BlockSpec((F, F2), lambda i: (0, 0)),
      ],
      out_specs=pl.BlockSpec((RB, F2), lambda i: (i, 0)),
      out_shape=jax.ShapeDtypeStruct((Np, F2), jnp.float32),
  )(p, h, dis, b, W)


def _tc_fin_body(p_ref, h_ref, dis_ref, b_ref, w_ref, bp_ref, o_ref):
  dis = dis_ref[...]
  z = (p_ref[0] + p_ref[1] + h_ref[...]) * dis + b_ref[...][None, :]
  g = jnp.where(z >= 0, z, 0.01 * z)
  o_ref[...] = jnp.dot(g, w_ref[...],
                       preferred_element_type=jnp.float32) + bp_ref[...][None, :]


def _tc_fin(p, h, dis, b, Wp, bp, N):
  F = h.shape[1]
  C = Wp.shape[1]
  rb = 400
  grid = (N // rb,)
  return pl.pallas_call(
      _tc_fin_body,
      grid=grid,
      in_specs=[
          pl.BlockSpec((2, rb, F), lambda i: (0, i, 0)),
          pl.BlockSpec((rb, F), lambda i: (i, 0)),
          pl.BlockSpec((rb, 1), lambda i: (i, 0)),
          pl.BlockSpec((F,), lambda i: (0,)),
          pl.BlockSpec((F, C), lambda i: (0, 0)),
          pl.BlockSpec((C,), lambda i: (0,)),
      ],
      out_specs=pl.BlockSpec((rb, C), lambda i: (i, 0)),
      out_shape=jax.ShapeDtypeStruct((N, C), jnp.float32),
  )(p, h, dis, b, Wp, bp)


@jax.jit
def kernel(x, edge_index, W1, b1, W2, b2, W3, b3, Wp, bp):
  N, D = x.shape
  E = edge_index.shape[1]
  Np = _pad_rows(N)

  epb = NTILES * B
  Ep = -(-E // epb) * epb
  pad_e = Ep - E
  nchunks = Ep // B

  src = jnp.concatenate(
      [edge_index[0], jnp.full((pad_e,), N, jnp.int32)]).reshape(nchunks, B)
  dst = jnp.concatenate(
      [edge_index[1], jnp.full((pad_e,), N, jnp.int32)]).reshape(nchunks, B)

  x_pad = jnp.pad(x, ((0, Np - N), (0, 0)))

  degp = _make_deg_kernel(Np, nchunks)(dst)
  h1, dis = _tc1(x_pad, W1, degp, Np)

  p1 = _make_prop_kernel(Np, nchunks, h1.shape[1])(h1, src, dst)
  h2 = _tc_mid(p1, h1, dis, b1, W2, Np)

  p2 = _make_prop_kernel(Np, nchunks, h2.shape[1])(h2, src, dst)
  h3 = _tc_mid(p2, h2, dis, b2, W3, Np)

  p3 = _make_prop_kernel(Np, nchunks, h3.shape[1])(h3, src, dst)
  out = _tc_fin(p3, h3, dis, b3, Wp, bp, N)
  return out


# speedup vs baseline: 29.3158x; 1.0024x over previous
import functools

import jax
import jax.numpy as jnp
from jax import lax
from jax.experimental import pallas as pl
from jax.experimental.pallas import tpu as pltpu
from jax.experimental.pallas import tpu_sc as plsc

B = 128
NTILES = 32
DEGW = 16


def _pad_rows(n):
  per = -(-n // NTILES)
  per = -(-per // 8) * 8
  return per * NTILES


def _make_deg_kernel(Np, nchunks):
  mesh = plsc.VectorSubcoreMesh(core_axis_name="c", subcore_axis_name="s")
  per_core = nchunks // 2
  per_tile = per_core // 16
  rows_per_tile = Np // NTILES * 2
  zrep = rows_per_tile // B

  @functools.partial(
      pl.kernel,
      mesh=mesh,
      out_type=jax.ShapeDtypeStruct((2, Np, DEGW), jnp.float32),
      compiler_params=pltpu.CompilerParams(use_tc_tiling_on_sc=False),
      scratch_types=[
          pltpu.VMEM((nchunks // NTILES, B), jnp.int32),
          pltpu.VMEM((B, DEGW), jnp.float32),
          pltpu.VMEM((B, DEGW), jnp.float32),
          pltpu.VMEM_SHARED((Np, DEGW), jnp.float32),
          pltpu.SemaphoreType.DMA,
      ],
  )
  def k(dst2d_hbm, out_hbm, dst_all, ones_v, stage_v, acc_sh, sem):
    c = lax.axis_index("c")
    s = lax.axis_index("s")
    base_chunk = c * per_core + s * per_tile
    pltpu.sync_copy(dst2d_hbm.at[pl.ds(base_chunk, per_tile)], dst_all)

    def fill(i, _):
      ones_v[i, :] = jnp.full((DEGW,), 1.0, jnp.float32)
      stage_v[i, :] = jnp.zeros((DEGW,), jnp.float32)
      return 0

    lax.fori_loop(0, B, fill, 0)

    base_row = s * rows_per_tile
    for r in range(zrep):
      pltpu.sync_copy(stage_v, acc_sh.at[pl.ds(base_row + r * B, B)])
    plsc.subcore_barrier()

    def body(t, _):
      pltpu.sync_copy(ones_v, acc_sh.at[dst_all.at[t]], add=True)
      return 0

    lax.fori_loop(0, per_tile, body, 0)
    plsc.subcore_barrier()

    for r in range(zrep):
      row = base_row + r * B
      pltpu.sync_copy(acc_sh.at[pl.ds(row, B)], stage_v)
      pltpu.sync_copy(stage_v, out_hbm.at[c, pl.ds(row, B)])

  return k


def _make_prop_kernel(Np, nchunks, F):
  mesh = plsc.VectorSubcoreMesh(core_axis_name="c", subcore_axis_name="s")
  per_core = nchunks // 2
  per_tile = per_core // 16
  rows_per_tile = Np // NTILES * 2
  zrep = rows_per_tile // B

  @functools.partial(
      pl.kernel,
      mesh=mesh,
      out_type=jax.ShapeDtypeStruct((2, Np, F), jnp.float32),
      compiler_params=pltpu.CompilerParams(use_tc_tiling_on_sc=False),
      scratch_types=[
          pltpu.VMEM((per_tile, B), jnp.int32),
          pltpu.VMEM((per_tile, B), jnp.int32),
          pltpu.VMEM((B, F), jnp.float32),
          pltpu.VMEM((B, F), jnp.float32),
          pltpu.VMEM_SHARED((Np, F), jnp.float32),
          pltpu.VMEM_SHARED((Np, F), jnp.float32),
          pltpu.SemaphoreType.DMA,
      ],
  )
  def k(h_hbm, src2d_hbm, dst2d_hbm, out_hbm, src_all, dst_all, rows0,
        stage_v, acc_sh, table_sh, g0):
    c = lax.axis_index("c")
    s = lax.axis_index("s")
    base_chunk = c * per_core + s * per_tile
    pltpu.sync_copy(src2d_hbm.at[pl.ds(base_chunk, per_tile)], src_all)
    pltpu.sync_copy(dst2d_hbm.at[pl.ds(base_chunk, per_tile)], dst_all)
    base_row0 = s * rows_per_tile
    for r in range(zrep):
      row = base_row0 + r * B
      pltpu.sync_copy(h_hbm.at[pl.ds(row, B)], table_sh.at[pl.ds(row, B)])

    def fill(i, _):
      for j in range(F // 16):
        stage_v[i, pl.ds(j * 16, 16)] = jnp.zeros((16,), jnp.float32)
      return 0

    lax.fori_loop(0, B, fill, 0)

    base_row = s * rows_per_tile
    for r in range(zrep):
      pltpu.sync_copy(stage_v, acc_sh.at[pl.ds(base_row + r * B, B)])
    plsc.subcore_barrier()

    def body(t, _):
      pltpu.async_copy(table_sh.at[src_all.at[t]], rows0, g0).wait()
      pltpu.async_copy(
          rows0, acc_sh.at[dst_all.at[t]], g0, add=True).wait()
      return 0

    lax.fori_loop(0, per_tile, body, 0)
    plsc.subcore_barrier()

    for r in range(zrep):
      row = base_row + r * B
      pltpu.sync_copy(acc_sh.at[pl.ds(row, B)], stage_v)
      pltpu.sync_copy(stage_v, out_hbm.at[c, pl.ds(row, B)])

  return k


RB = 512


def _tc1_body(x_ref, w_ref, deg_ref, h_ref, dis_ref):
  deg = 1.0 + deg_ref[0, :, 0] + deg_ref[1, :, 0]
  dis = lax.rsqrt(deg)
  h = jnp.dot(x_ref[...], w_ref[...], preferred_element_type=jnp.float32)
  h_ref[...] = h * dis[:, None]
  dis_ref[...] = dis[:, None]


def _tc1(x, W1, degp, Np):
  D = x.shape[1]
  F = W1.shape[1]
  grid = (Np // RB,)
  return pl.pallas_call(
      _tc1_body,
      grid=grid,
      in_specs=[
          pl.BlockSpec((RB, D), lambda i: (i, 0)),
          pl.BlockSpec((D, F), lambda i: (0, 0)),
          pl.BlockSpec((2, RB, DEGW), lambda i: (0, i, 0)),
      ],
      out_specs=[
          pl.BlockSpec((RB, F), lambda i: (i, 0)),
          pl.BlockSpec((RB, 1), lambda i: (i, 0)),
      ],
      out_shape=[
          jax.ShapeDtypeStruct((Np, F), jnp.float32),
          jax.ShapeDtypeStruct((Np, 1), jnp.float32),
      ],
  )(x, W1, degp)


def _tc_mid_body(p_ref, h_ref, dis_ref, b_ref, w_ref, o_ref):
  dis = dis_ref[...]
  z = (p_ref[0] + p_ref[1] + h_ref[...]) * dis + b_ref[...][None, :]
  g = jnp.where(z >= 0, z, 0.01 * z)
  o_ref[...] = jnp.dot(g, w_ref[...],
                       preferred_element_type=jnp.float32) * dis


def _tc_mid(p, h, dis, b, W, Np):
  F = h.shape[1]
  F2 = W.shape[1]
  grid = (Np // RB,)
  return pl.pallas_call(
      _tc_mid_body,
      grid=grid,
      in_specs=[
          pl.BlockSpec((2, RB, F), lambda i: (0, i, 0)),
          pl.BlockSpec((RB, F), lambda i: (i, 0)),
          pl.BlockSpec((RB, 1), lambda i: (i, 0)),
          pl.BlockSpec((F,), lambda i: (0,)),
          pl.BlockSpec((F, F2), lambda i: (0, 0)),
      ],
      out_specs=pl.BlockSpec((RB, F2), lambda i: (i, 0)),
      out_shape=jax.ShapeDtypeStruct((Np, F2), jnp.float32),
  )(p, h, dis, b, W)


def _tc_fin_body(p_ref, h_ref, dis_ref, b_ref, w_ref, bp_ref, o_ref):
  dis = dis_ref[...]
  z = (p_ref[0] + p_ref[1] + h_ref[...]) * dis + b_ref[...][None, :]
  g = jnp.where(z >= 0, z, 0.01 * z)
  o_ref[...] = jnp.dot(g, w_ref[...],
                       preferred_element_type=jnp.float32) + bp_ref[...][None, :]


def _tc_fin(p, h, dis, b, Wp, bp, N):
  F = h.shape[1]
  C = Wp.shape[1]
  rb = 400
  grid = (N // rb,)
  return pl.pallas_call(
      _tc_fin_body,
      grid=grid,
      in_specs=[
          pl.BlockSpec((2, rb, F), lambda i: (0, i, 0)),
          pl.BlockSpec((rb, F), lambda i: (i, 0)),
          pl.BlockSpec((rb, 1), lambda i: (i, 0)),
          pl.BlockSpec((F,), lambda i: (0,)),
          pl.BlockSpec((F, C), lambda i: (0, 0)),
          pl.BlockSpec((C,), lambda i: (0,)),
      ],
      out_specs=pl.BlockSpec((rb, C), lambda i: (i, 0)),
      out_shape=jax.ShapeDtypeStruct((N, C), jnp.float32),
  )(p, h, dis, b, Wp, bp)


@jax.jit
def kernel(x, edge_index, W1, b1, W2, b2, W3, b3, Wp, bp):
  N, D = x.shape
  E = edge_index.shape[1]
  Np = _pad_rows(N)

  epb = NTILES * B
  Ep = -(-E // epb) * epb
  pad_e = Ep - E
  nchunks = Ep // B

  src = jnp.concatenate(
      [edge_index[0], jnp.full((pad_e,), N, jnp.int32)]).reshape(nchunks, B)
  dst = jnp.concatenate(
      [edge_index[1], jnp.full((pad_e,), N, jnp.int32)]).reshape(nchunks, B)

  degp = _make_deg_kernel(Np, nchunks)(dst)
  h1, dis = _tc1(x, W1, degp, Np)

  p1 = _make_prop_kernel(Np, nchunks, h1.shape[1])(h1, src, dst)
  h2 = _tc_mid(p1, h1, dis, b1, W2, Np)

  p2 = _make_prop_kernel(Np, nchunks, h2.shape[1])(h2, src, dst)
  h3 = _tc_mid(p2, h2, dis, b2, W3, Np)

  p3 = _make_prop_kernel(Np, nchunks, h3.shape[1])(h3, src, dst)
  out = _tc_fin(p3, h3, dis, b3, Wp, bp, N)
  return out


# speedup vs baseline: 31.9535x; 1.0900x over previous
import functools

import jax
import jax.numpy as jnp
from jax import lax
from jax.experimental import pallas as pl
from jax.experimental.pallas import tpu as pltpu
from jax.experimental.pallas import tpu_sc as plsc

B = 128
NTILES = 32
DEGW = 16


def _pad_rows(n):
  per = -(-n // NTILES)
  per = -(-per // 8) * 8
  return per * NTILES


def _make_deg_kernel(Np, nchunks):
  mesh = plsc.VectorSubcoreMesh(core_axis_name="c", subcore_axis_name="s")
  per_core = nchunks // 2
  per_tile = per_core // 16
  rows_per_tile = Np // NTILES * 2
  zrep = rows_per_tile // B

  @functools.partial(
      pl.kernel,
      mesh=mesh,
      out_type=jax.ShapeDtypeStruct((2, Np, DEGW), jnp.float32),
      compiler_params=pltpu.CompilerParams(use_tc_tiling_on_sc=False),
      scratch_types=[
          pltpu.VMEM((nchunks // NTILES, B), jnp.int32),
          pltpu.VMEM((B, DEGW), jnp.float32),
          pltpu.VMEM((B, DEGW), jnp.float32),
          pltpu.VMEM_SHARED((Np, DEGW), jnp.float32),
          pltpu.SemaphoreType.DMA,
      ],
  )
  def k(dst2d_hbm, out_hbm, dst_all, ones_v, stage_v, acc_sh, sem):
    c = lax.axis_index("c")
    s = lax.axis_index("s")
    base_chunk = c * per_core + s * per_tile
    pltpu.sync_copy(dst2d_hbm.at[pl.ds(base_chunk, per_tile)], dst_all)

    def fill(i, _):
      ones_v[i, :] = jnp.full((DEGW,), 1.0, jnp.float32)
      stage_v[i, :] = jnp.zeros((DEGW,), jnp.float32)
      return 0

    lax.fori_loop(0, B, fill, 0)

    base_row = s * rows_per_tile
    for r in range(zrep):
      pltpu.sync_copy(stage_v, acc_sh.at[pl.ds(base_row + r * B, B)])
    plsc.subcore_barrier()

    def body(t, _):
      pltpu.sync_copy(ones_v, acc_sh.at[dst_all.at[t]], add=True)
      return 0

    lax.fori_loop(0, per_tile, body, 0)
    plsc.subcore_barrier()

    for r in range(zrep):
      row = base_row + r * B
      pltpu.sync_copy(acc_sh.at[pl.ds(row, B)], stage_v)
      pltpu.sync_copy(stage_v, out_hbm.at[c, pl.ds(row, B)])

  return k


def _make_prop_kernel(Np, nchunks, F):
  mesh = plsc.VectorSubcoreMesh(core_axis_name="c", subcore_axis_name="s")
  per_core = nchunks // 2
  per_tile = per_core // 16
  rows_per_tile = Np // NTILES * 2
  zrep = rows_per_tile // B

  @functools.partial(
      pl.kernel,
      mesh=mesh,
      out_type=jax.ShapeDtypeStruct((2, Np, F), jnp.float32),
      compiler_params=pltpu.CompilerParams(use_tc_tiling_on_sc=False),
      scratch_types=[
          pltpu.VMEM((per_tile, B), jnp.int32),
          pltpu.VMEM((per_tile, B), jnp.int32),
          pltpu.VMEM((B, F), jnp.float32),
          pltpu.VMEM((B, F), jnp.float32),
          pltpu.VMEM_SHARED((Np, F), jnp.float32),
          pltpu.VMEM_SHARED((Np, F), jnp.float32),
          pltpu.SemaphoreType.DMA,
      ],
  )
  def k(h_hbm, src2d_hbm, dst2d_hbm, out_hbm, src_all, dst_all, rows0,
        stage_v, acc_sh, table_sh, g0):
    c = lax.axis_index("c")
    s = lax.axis_index("s")
    base_chunk = c * per_core + s * per_tile
    pltpu.sync_copy(src2d_hbm.at[pl.ds(base_chunk, per_tile)], src_all)
    pltpu.sync_copy(dst2d_hbm.at[pl.ds(base_chunk, per_tile)], dst_all)
    base_row0 = s * rows_per_tile
    for r in range(zrep):
      row = base_row0 + r * B
      pltpu.sync_copy(h_hbm.at[pl.ds(row, B)], table_sh.at[pl.ds(row, B)])

    def fill(i, _):
      for j in range(F // 16):
        stage_v[i, pl.ds(j * 16, 16)] = jnp.zeros((16,), jnp.float32)
      return 0

    lax.fori_loop(0, B, fill, 0)

    base_row = s * rows_per_tile
    for r in range(zrep):
      pltpu.sync_copy(stage_v, acc_sh.at[pl.ds(base_row + r * B, B)])
    plsc.subcore_barrier()

    def body(t, _):
      pltpu.async_copy(table_sh.at[src_all.at[t]], rows0, g0).wait()
      pltpu.async_copy(
          rows0, acc_sh.at[dst_all.at[t]], g0, add=True).wait()
      return 0

    lax.fori_loop(0, per_tile, body, 0)
    plsc.subcore_barrier()

    for r in range(zrep):
      row = base_row + r * B
      pltpu.sync_copy(acc_sh.at[pl.ds(row, B)], stage_v)
      pltpu.sync_copy(stage_v, out_hbm.at[c, pl.ds(row, B)])

  return k


RB = 2048


def _tc1_body(x_ref, w_ref, deg_ref, h_ref, dis_ref):
  deg = 1.0 + deg_ref[0, :, 0] + deg_ref[1, :, 0]
  dis = lax.rsqrt(deg)
  h = jnp.dot(x_ref[...], w_ref[...], preferred_element_type=jnp.float32)
  h_ref[...] = h * dis[:, None]
  dis_ref[...] = dis[:, None]


def _tc1(x, W1, degp, Np):
  D = x.shape[1]
  F = W1.shape[1]
  grid = (Np // RB,)
  return pl.pallas_call(
      _tc1_body,
      grid=grid,
      in_specs=[
          pl.BlockSpec((RB, D), lambda i: (i, 0)),
          pl.BlockSpec((D, F), lambda i: (0, 0)),
          pl.BlockSpec((2, RB, DEGW), lambda i: (0, i, 0)),
      ],
      out_specs=[
          pl.BlockSpec((RB, F), lambda i: (i, 0)),
          pl.BlockSpec((RB, 1), lambda i: (i, 0)),
      ],
      out_shape=[
          jax.ShapeDtypeStruct((Np, F), jnp.float32),
          jax.ShapeDtypeStruct((Np, 1), jnp.float32),
      ],
  )(x, W1, degp)


def _tc_mid_body(p_ref, h_ref, dis_ref, b_ref, w_ref, o_ref):
  dis = dis_ref[...]
  z = (p_ref[0] + p_ref[1] + h_ref[...]) * dis + b_ref[...][None, :]
  g = jnp.where(z >= 0, z, 0.01 * z)
  o_ref[...] = jnp.dot(g, w_ref[...],
                       preferred_element_type=jnp.float32) * dis


def _tc_mid(p, h, dis, b, W, Np):
  F = h.shape[1]
  F2 = W.shape[1]
  grid = (Np // RB,)
  return pl.pallas_call(
      _tc_mid_body,
      grid=grid,
      in_specs=[
          pl.BlockSpec((2, RB, F), lambda i: (0, i, 0)),
          pl.BlockSpec((RB, F), lambda i: (i, 0)),
          pl.BlockSpec((RB, 1), lambda i: (i, 0)),
          pl.BlockSpec((F,), lambda i: (0,)),
          pl.BlockSpec((F, F2), lambda i: (0, 0)),
      ],
      out_specs=pl.BlockSpec((RB, F2), lambda i: (i, 0)),
      out_shape=jax.ShapeDtypeStruct((Np, F2), jnp.float32),
  )(p, h, dis, b, W)


def _tc_fin_body(p_ref, h_ref, dis_ref, b_ref, w_ref, bp_ref, o_ref):
  dis = dis_ref[...]
  z = (p_ref[0] + p_ref[1] + h_ref[...]) * dis + b_ref[...][None, :]
  g = jnp.where(z >= 0, z, 0.01 * z)
  o_ref[...] = jnp.dot(g, w_ref[...],
                       preferred_element_type=jnp.float32) + bp_ref[...][None, :]


def _tc_fin(p, h, dis, b, Wp, bp, N):
  F = h.shape[1]
  C = Wp.shape[1]
  rb = 2000
  grid = (N // rb,)
  return pl.pallas_call(
      _tc_fin_body,
      grid=grid,
      in_specs=[
          pl.BlockSpec((2, rb, F), lambda i: (0, i, 0)),
          pl.BlockSpec((rb, F), lambda i: (i, 0)),
          pl.BlockSpec((rb, 1), lambda i: (i, 0)),
          pl.BlockSpec((F,), lambda i: (0,)),
          pl.BlockSpec((F, C), lambda i: (0, 0)),
          pl.BlockSpec((C,), lambda i: (0,)),
      ],
      out_specs=pl.BlockSpec((rb, C), lambda i: (i, 0)),
      out_shape=jax.ShapeDtypeStruct((N, C), jnp.float32),
  )(p, h, dis, b, Wp, bp)


@jax.jit
def kernel(x, edge_index, W1, b1, W2, b2, W3, b3, Wp, bp):
  N, D = x.shape
  E = edge_index.shape[1]
  Np = _pad_rows(N)

  epb = NTILES * B
  Ep = -(-E // epb) * epb
  pad_e = Ep - E
  nchunks = Ep // B

  src = jnp.concatenate(
      [edge_index[0], jnp.full((pad_e,), N, jnp.int32)]).reshape(nchunks, B)
  dst = jnp.concatenate(
      [edge_index[1], jnp.full((pad_e,), N, jnp.int32)]).reshape(nchunks, B)

  degp = _make_deg_kernel(Np, nchunks)(dst)
  h1, dis = _tc1(x, W1, degp, Np)

  p1 = _make_prop_kernel(Np, nchunks, h1.shape[1])(h1, src, dst)
  h2 = _tc_mid(p1, h1, dis, b1, W2, Np)

  p2 = _make_prop_kernel(Np, nchunks, h2.shape[1])(h2, src, dst)
  h3 = _tc_mid(p2, h2, dis, b2, W3, Np)

  p3 = _make_prop_kernel(Np, nchunks, h3.shape[1])(h3, src, dst)
  out = _tc_fin(p3, h3, dis, b3, Wp, bp, N)
  return out
